# baseline (speedup 1.0000x reference)
import jax
import jax.numpy as jnp
from jax import lax
from jax.experimental import pallas as pl
from jax.experimental.pallas import tpu as pltpu
from jax.experimental.pallas import tpu_sc as plsc

NUM_NODES = 10000
NUM_HEADS = 8
BATCH = 1024
SEQ_LEN = 50

PAIRS = SEQ_LEN * SEQ_LEN
OUT_ROW = PAIRS * NUM_HEADS
IDX_ROWS = 20
IDX_COLS = 128
NUM_WORKERS = 32
B_PER_W = BATCH // NUM_WORKERS
SEQ_BLOCK = B_PER_W * SEQ_LEN
EXP_CHUNKS = OUT_ROW // 16
EXP_UNROLL = 10


def _body(seq_hbm, table_hbm, out_hbm, seq_v, idx_v, vals_v, out_v, sem):
    wid = lax.axis_index("s") * 2 + lax.axis_index("c")
    lanes = lax.iota(jnp.int32, 16)
    rep = lanes // 8
    zeros16 = jnp.zeros((16,), jnp.int32)
    tail_mask = lanes < 2

    row19 = zeros16 + 19
    for z in range(4):
        plsc.store_scatter(idx_v, [row19, 64 + z * 16 + lanes], zeros16)

    pltpu.sync_copy(
        seq_hbm.at[pl.ds(wid * SEQ_BLOCK, SEQ_BLOCK)],
        seq_v.at[pl.ds(0, SEQ_BLOCK)],
    )

    def per_batch(k, carry):
        b = wid * B_PER_W + k
        s0 = k * SEQ_LEN

        cols = [plsc.load_gather(seq_v, [s0 + cj * 16 + lanes]) for cj in range(4)]

        def per_i(i, c):
            base_addr = plsc.load_gather(seq_v, [zeros16 + (s0 + i)]) * NUM_NODES
            p0 = i * SEQ_LEN
            for cj in range(4):
                vec = c[cj] + base_addr
                pos = p0 + cj * 16 + lanes
                mask = tail_mask if cj == 3 else None
                plsc.store_scatter(
                    idx_v, [pos // IDX_COLS, pos % IDX_COLS], vec, mask=mask
                )
            return c

        lax.fori_loop(0, SEQ_LEN, per_i, tuple(cols))

        copies = [
            pltpu.async_copy(
                table_hbm.at[idx_v.at[ck]],
                vals_v.at[pl.ds(ck * IDX_COLS, IDX_COLS)],
                sem,
            )
            for ck in range(IDX_ROWS)
        ]
        for c in copies:
            c.wait()

        def expand(c0, c):
            for u in range(EXP_UNROLL):
                cc = c0 * EXP_UNROLL + u
                g = plsc.load_gather(vals_v, [rep + 2 * cc])
                out_v[pl.ds(16 * cc, 16)] = g
            return c

        lax.fori_loop(0, EXP_CHUNKS // EXP_UNROLL, expand, 0)

        pltpu.sync_copy(out_v, out_hbm.at[b])
        return carry

    lax.fori_loop(0, B_PER_W, per_batch, 0)


@jax.jit
def kernel(user_seq, spatial_pos_table):
    seq_flat = user_seq.reshape(BATCH * SEQ_LEN)
    table_flat = spatial_pos_table.reshape(NUM_NODES * NUM_NODES)
    k = pl.kernel(
        _body,
        out_type=jax.ShapeDtypeStruct((BATCH, OUT_ROW), jnp.int32),
        mesh=plsc.VectorSubcoreMesh(core_axis_name="c", subcore_axis_name="s"),
        scratch_types=[
            pltpu.VMEM((SEQ_BLOCK + 64,), jnp.int32),
            pltpu.VMEM((IDX_ROWS, IDX_COLS), jnp.int32),
            pltpu.VMEM((IDX_ROWS * IDX_COLS,), jnp.int32),
            pltpu.VMEM((OUT_ROW,), jnp.int32),
            pltpu.SemaphoreType.DMA,
        ],
        compiler_params=pltpu.CompilerParams(needs_layout_passes=False),
    )
    out = k(seq_flat, table_flat)
    return out.reshape(BATCH, SEQ_LEN, SEQ_LEN, NUM_HEADS)
